# baseline (speedup 1.0000x reference)
import contextlib
import functools

import jax
import jax.numpy as jnp
from jax import lax
from jax.experimental import pallas as pl
from jax.experimental.pallas import tpu as pltpu
from jax.experimental.pallas import tpu_sc as plsc

_N_CLS = 1000
_N_PAD = 1024
_FEAT = 128
_BATCH = 4096
_M = 0.95
_TEMP = 0.1

_NSC = 2
_NTILE = 16
_CHUNK = _BATCH // _NTILE
_CLS_PER_TILE = _N_PAD // (_NSC * _NTILE)
_WIN = 128
_NBIG = 4


def _lshuf(v, idx):
    return v.at[idx].get(mode="promise_in_bounds")


def _cumsum16(v, iota16):
    for d in (1, 2, 4, 8):
        sh = _lshuf(v, jnp.maximum(iota16 - d, 0))
        v = v + jnp.where(iota16 >= d, sh, 0)
    return v


def _allsum16(v, iota16):
    for d in (1, 2, 4, 8):
        v = v + _lshuf(v, iota16 ^ d)
    return v


def _sget(ref, idx):
    return ref[pl.ds(idx, 16)][0]


def _sc_body(feat_hbm, lbl_hbm, prot_hbm, out_hbm,
             counts_sh, sortedidx_sh,
             lbl_v, cnt_v, rank_v, all16_v, base_v, excl_v, tot_v,
             pos_v, val_v, idxwin_v, idxbig_v, featall_v,
             prot_v, zero_v, sem):
    core = lax.axis_index("c")
    s = lax.axis_index("s")
    zeros16i = jnp.zeros((16,), jnp.int32)

    _stk = contextlib.ExitStack()
    _stk.enter_context(jax.named_scope("phA"))
    pltpu.sync_copy(lbl_hbm.at[pl.ds(s * _CHUNK, _CHUNK)],
                    lbl_v.at[pl.ds(0, _CHUNK)])

    def zero_body(i, c):
        cnt_v[pl.ds(i * 16, 16)] = zeros16i
        return c
    lax.fori_loop(0, (_N_PAD + 128) // 16, zero_body, 0)

    iota16 = lax.iota(jnp.int32, 16)

    def hist_body(i, acc):
        l = _sget(lbl_v, i)
        b = (l // 16) * 16
        msk = iota16 == jnp.full((16,), l - b, jnp.int32)
        v = cnt_v[pl.ds(b, 16)]
        rvec = _allsum16(jnp.where(msk, v, 0), iota16)
        cnt_v[pl.ds(b, 16)] = jnp.where(msk, v + 1, v)
        acc = jnp.where(iota16 == jnp.full((16,), i % 16, jnp.int32),
                        rvec, acc)

        @pl.when(i % 16 == 15)
        def _():
            rank_v[pl.ds((i // 16) * 16, 16)] = acc
        return acc
    lax.fori_loop(0, _CHUNK, hist_body, zeros16i)
    pltpu.sync_copy(cnt_v.at[pl.ds(0, _N_PAD)], counts_sh.at[s])

    @pl.when(s == 0)
    def _():
        for k in range(_WIN // 16):
            zero_v[pl.ds(k * 16, 16)] = zeros16i
        for b in range(_NBIG):
            pltpu.sync_copy(zero_v,
                            sortedidx_sh.at[pl.ds(_BATCH + b * _WIN, _WIN)])

    _stk.close()
    _stk.enter_context(jax.named_scope("phB"))
    plsc.subcore_barrier()

    pltpu.sync_copy(counts_sh, all16_v)

    def chunk_body(k, carry):
        tot = zeros16i
        below = zeros16i
        for w in range(_NTILE):
            v = all16_v[w, pl.ds(k * 16, 16)]
            tot = tot + v
            ind = jnp.where(w < s, 1, 0)
            below = below + v * jnp.full((16,), ind, jnp.int32)
        cs = _cumsum16(tot, iota16)
        excl = cs - tot + jnp.full((16,), carry, jnp.int32)
        base_v[pl.ds(k * 16, 16)] = excl + below
        excl_v[pl.ds(k * 16, 16)] = excl
        tot_v[pl.ds(k * 16, 16)] = tot
        return carry + cs[15]
    lax.fori_loop(0, _N_PAD // 16, chunk_body, jnp.int32(0))

    _stk.close()
    _stk.enter_context(jax.named_scope("phC"))
    def pos_body(g, c):
        posv = zeros16i
        for j in range(16):
            i = g * 16 + j
            l = _sget(lbl_v, i)
            pz = _sget(base_v, l) + _sget(rank_v, i)
            posv = jnp.where(iota16 == j, jnp.full((16,), pz, jnp.int32),
                             posv)
        pos_v[g // 8, pl.ds((g % 8) * 16, 16)] = posv
        return c
    lax.fori_loop(0, _CHUNK // 16, pos_body, 0)

    for j in range(_CHUNK // 128):
        for k in range(8):
            val_v[j, pl.ds(k * 16, 16)] = (
                lax.iota(jnp.int32, 16) + (s * _CHUNK + j * 128 + k * 16))
    for j in range(_CHUNK // 128):
        pltpu.sync_copy(val_v.at[j], sortedidx_sh.at[pos_v.at[j]])

    _stk.close()
    _stk.enter_context(jax.named_scope("phD"))
    plsc.subcore_barrier()

    cbase = core * (_N_PAD // _NSC) + s * _CLS_PER_TILE
    ntail = _N_CLS - (_N_PAD - _CLS_PER_TILE)

    @pl.when(cbase != _N_PAD - _CLS_PER_TILE)
    def _():
        pltpu.sync_copy(prot_hbm.at[pl.ds(cbase, _CLS_PER_TILE)], prot_v)

    @pl.when(cbase == _N_PAD - _CLS_PER_TILE)
    def _():
        pltpu.sync_copy(prot_hbm.at[pl.ds(_N_PAD - _CLS_PER_TILE, ntail)],
                        prot_v.at[pl.ds(0, ntail)])

    tile_start = _sget(excl_v, cbase)
    bigbase = (tile_start // _WIN) * _WIN
    for b in range(_NBIG):
        pltpu.sync_copy(sortedidx_sh.at[pl.ds(bigbase + b * _WIN, _WIN)],
                        idxbig_v.at[pl.ds(b * _WIN, _WIN)])
    _descs = [
        pltpu.async_copy(feat_hbm.at[idxbig_v.at[pl.ds(b * _WIN, _WIN)]],
                         featall_v.at[pl.ds(b * _WIN, _WIN)], sem)
        for b in range(_NBIG)]
    for _d in _descs:
        _d.wait()

    def class_body(cl, win_lo):
        c = cbase + cl
        n_c = _sget(tot_v, c)
        start_c = _sget(excl_v, c)
        p = tuple(prot_v[cl, pl.ds(k * 16, 16)] for k in range(8))

        def samp_body(j, carry):
            wlo = carry[0]
            pk = carry[1:]
            row = start_c + j

            def reload(_):
                nl = (row // _WIN) * _WIN
                pltpu.sync_copy(sortedidx_sh.at[pl.ds(nl, _WIN)], idxwin_v)
                pltpu.async_copy(
                    feat_hbm.at[idxwin_v],
                    featall_v.at[pl.ds(_NBIG * _WIN, _WIN)], sem).wait()
                return nl

            inbig = row < bigbase + _NBIG * _WIN
            wlo = lax.cond((~inbig) & (row >= wlo + _WIN), reload,
                           lambda _: wlo, 0)
            off = jnp.where(inbig, row - bigbase,
                            _NBIG * _WIN + (row - wlo))
            f = tuple(featall_v[off, pl.ds(k * 16, 16)] for k in range(8))
            new = tuple(pk[k] * _M + f[k] * (1.0 - _M) for k in range(8))
            acc = new[0] * new[0]
            for k in range(1, 8):
                acc = acc + new[k] * new[k]
            ss = jnp.maximum(_allsum16(acc, iota16)[0], jnp.float32(1e-24))
            yi = jnp.int32(0x5F3759DF) - lax.shift_right_logical(
                lax.bitcast_convert_type(ss, jnp.int32), 1)
            y = lax.bitcast_convert_type(yi, jnp.float32)
            for _ in range(3):
                y = y * (1.5 - 0.5 * ss * y * y)
            y = jnp.minimum(y, jnp.float32(1e12))
            yv = jnp.full((16,), y, jnp.float32)
            new = tuple(n * yv for n in new)
            return (wlo,) + new

        carry = lax.fori_loop(0, n_c, samp_body, (win_lo,) + p)
        for k in range(8):
            prot_v[cl, pl.ds(k * 16, 16)] = carry[1 + k]
        return carry[0]

    lax.fori_loop(0, _CLS_PER_TILE, class_body, jnp.int32(-(2 ** 30)))

    _stk.close()
    @pl.when(cbase != _N_PAD - _CLS_PER_TILE)
    def _():
        pltpu.sync_copy(prot_v, out_hbm.at[pl.ds(cbase, _CLS_PER_TILE)])

    @pl.when(cbase == _N_PAD - _CLS_PER_TILE)
    def _():
        pltpu.sync_copy(prot_v.at[pl.ds(0, ntail)],
                        out_hbm.at[pl.ds(_N_PAD - _CLS_PER_TILE, ntail)])


@functools.partial(
    pl.kernel,
    out_type=jax.ShapeDtypeStruct((_N_CLS, _FEAT), jnp.float32),
    mesh=plsc.VectorSubcoreMesh(core_axis_name="c", subcore_axis_name="s",
                                num_cores=_NSC, num_subcores=_NTILE),
    scratch_types=[
        pltpu.VMEM_SHARED((_NTILE, _N_PAD), jnp.int32),
        pltpu.VMEM_SHARED((_BATCH + _NBIG * _WIN,), jnp.int32),
        pltpu.VMEM((_CHUNK + 128,), jnp.int32),
        pltpu.VMEM((_N_PAD + 128,), jnp.int32),
        pltpu.VMEM((_CHUNK + 128,), jnp.int32),
        pltpu.VMEM((_NTILE, _N_PAD), jnp.int32),
        pltpu.VMEM((_N_PAD + 128,), jnp.int32),
        pltpu.VMEM((_N_PAD + 128,), jnp.int32),
        pltpu.VMEM((_N_PAD + 128,), jnp.int32),
        pltpu.VMEM((_CHUNK // 128, 128), jnp.int32),
        pltpu.VMEM((_CHUNK // 128, 128), jnp.int32),
        pltpu.VMEM((_WIN,), jnp.int32),
        pltpu.VMEM((_NBIG * _WIN,), jnp.int32),
        pltpu.VMEM(((_NBIG + 1) * _WIN, _FEAT), jnp.float32),
        pltpu.VMEM((_CLS_PER_TILE, _FEAT), jnp.float32),
        pltpu.VMEM((_WIN,), jnp.int32),
        pltpu.SemaphoreType.DMA,
    ],
)
def _sc_update(feat_hbm, lbl_hbm, prot_hbm, out_hbm, *scratch):
    _sc_body(feat_hbm, lbl_hbm, prot_hbm, out_hbm, *scratch)


def _loss_body(proto_ref, out_ref):
    p = proto_ref[...]
    logits = jax.lax.dot_general(
        p, p, (((1,), (1,)), ((), ())),
        preferred_element_type=jnp.float32) * (1.0 / _TEMP)
    row = jax.lax.broadcasted_iota(jnp.int32, (_N_CLS, _N_CLS), 0)
    col = jax.lax.broadcasted_iota(jnp.int32, (_N_CLS, _N_CLS), 1)
    offdiag = (row != col).astype(jnp.float32)
    num = jnp.sum(jnp.exp(logits) * offdiag, axis=1)
    mean_prob_neg = jnp.log(num * (1.0 / (_N_CLS - 1)))
    out_ref[0, 0] = jnp.sum(mean_prob_neg) * (1.0 / _N_CLS)


def kernel(features, labels, prototypes):
    updated = _sc_update(features, labels.astype(jnp.int32), prototypes)

    loss = pl.pallas_call(
        _loss_body,
        out_shape=jax.ShapeDtypeStruct((1, 1), jnp.float32),
        in_specs=[pl.BlockSpec(memory_space=pltpu.VMEM)],
        out_specs=pl.BlockSpec(memory_space=pltpu.SMEM),
    )(updated)
    return loss[0, 0]


# speedup vs baseline: 1.5567x; 1.5567x over previous
import contextlib
import functools

import jax
import jax.numpy as jnp
from jax import lax
from jax.experimental import pallas as pl
from jax.experimental.pallas import tpu as pltpu
from jax.experimental.pallas import tpu_sc as plsc

_N_CLS = 1000
_N_PAD = 1024
_FEAT = 128
_BATCH = 4096
_M = 0.95
_TEMP = 0.1

_NSC = 2
_NTILE = 16
_CHUNK = _BATCH // _NTILE
_CLS_PER_TILE = _N_PAD // (_NSC * _NTILE)
_WIN = 128


def _lshuf(v, idx):
    return v.at[idx].get(mode="promise_in_bounds")


def _cumsum16(v, iota16):
    for d in (1, 2, 4, 8):
        sh = _lshuf(v, jnp.maximum(iota16 - d, 0))
        v = v + jnp.where(iota16 >= d, sh, 0)
    return v


def _allsum16(v, iota16):
    for d in (1, 2, 4, 8):
        v = v + _lshuf(v, iota16 ^ d)
    return v


def _sget(ref, idx):
    return ref[pl.ds(idx, 16)][0]


def _sc_body(feat_hbm, lbl_hbm, prot_hbm, out_hbm,
             counts_sh, sortedidx_sh,
             lbl_v, cnt_v, rank_v, all16_v, base_v, excl_v, tot_v,
             pos_v, val_v, idxwin_v, featwin_v, prot_v, zero_v, sem):
    core = lax.axis_index("c")
    s = lax.axis_index("s")
    zeros16i = jnp.zeros((16,), jnp.int32)

    _stk = contextlib.ExitStack()
    _stk.enter_context(jax.named_scope("phA"))
    pltpu.sync_copy(lbl_hbm.at[pl.ds(s * _CHUNK, _CHUNK)],
                    lbl_v.at[pl.ds(0, _CHUNK)])

    def zero_body(i, c):
        cnt_v[pl.ds(i * 16, 16)] = zeros16i
        return c
    lax.fori_loop(0, (_N_PAD + 128) // 16, zero_body, 0)

    iota16 = lax.iota(jnp.int32, 16)

    def hist_one(i, acc):
        l = _sget(lbl_v, i)
        b = (l // 16) * 16
        msk = iota16 == jnp.full((16,), l - b, jnp.int32)
        v = cnt_v[pl.ds(b, 16)]
        rvec = _allsum16(jnp.where(msk, v, 0), iota16)
        cnt_v[pl.ds(b, 16)] = jnp.where(msk, v + 1, v)
        return jnp.where(iota16 == jnp.full((16,), i % 16, jnp.int32),
                         rvec, acc)

    def hist_body(ii, acc):
        acc = hist_one(2 * ii, acc)
        acc = hist_one(2 * ii + 1, acc)

        @pl.when(ii % 8 == 7)
        def _():
            rank_v[pl.ds((ii // 8) * 16, 16)] = acc
        return acc
    lax.fori_loop(0, _CHUNK // 2, hist_body, zeros16i)
    pltpu.sync_copy(cnt_v.at[pl.ds(0, _N_PAD)], counts_sh.at[s])

    @pl.when(s == 0)
    def _():
        for k in range(_WIN // 16):
            zero_v[pl.ds(k * 16, 16)] = zeros16i
        pltpu.sync_copy(zero_v, sortedidx_sh.at[pl.ds(_BATCH, _WIN)])

    _stk.close()
    _stk.enter_context(jax.named_scope("phB"))
    plsc.subcore_barrier()

    pltpu.sync_copy(counts_sh, all16_v)

    def chunk_body(k, carry):
        tot = zeros16i
        below = zeros16i
        for w in range(_NTILE):
            v = all16_v[w, pl.ds(k * 16, 16)]
            tot = tot + v
            ind = jnp.where(w < s, 1, 0)
            below = below + v * jnp.full((16,), ind, jnp.int32)
        cs = _cumsum16(tot, iota16)
        excl = cs - tot + jnp.full((16,), carry, jnp.int32)
        base_v[pl.ds(k * 16, 16)] = excl + below
        excl_v[pl.ds(k * 16, 16)] = excl
        tot_v[pl.ds(k * 16, 16)] = tot
        return carry + cs[15]
    lax.fori_loop(0, _N_PAD // 16, chunk_body, jnp.int32(0))

    _stk.close()
    _stk.enter_context(jax.named_scope("phC"))
    def pos_body(g, c):
        posv = zeros16i
        for j in range(16):
            i = g * 16 + j
            l = _sget(lbl_v, i)
            pz = _sget(base_v, l) + _sget(rank_v, i)
            posv = jnp.where(iota16 == j, jnp.full((16,), pz, jnp.int32),
                             posv)
        pos_v[g // 8, pl.ds((g % 8) * 16, 16)] = posv
        return c
    lax.fori_loop(0, _CHUNK // 16, pos_body, 0)

    for j in range(_CHUNK // 128):
        for k in range(8):
            val_v[j, pl.ds(k * 16, 16)] = (
                lax.iota(jnp.int32, 16) + (s * _CHUNK + j * 128 + k * 16))
    for j in range(_CHUNK // 128):
        pltpu.sync_copy(val_v.at[j], sortedidx_sh.at[pos_v.at[j]])

    _stk.close()
    _stk.enter_context(jax.named_scope("phD"))
    plsc.subcore_barrier()

    cbase = core * (_N_PAD // _NSC) + s * _CLS_PER_TILE
    ntail = _N_CLS - (_N_PAD - _CLS_PER_TILE)

    @pl.when(cbase != _N_PAD - _CLS_PER_TILE)
    def _():
        pltpu.sync_copy(prot_hbm.at[pl.ds(cbase, _CLS_PER_TILE)], prot_v)

    @pl.when(cbase == _N_PAD - _CLS_PER_TILE)
    def _():
        pltpu.sync_copy(prot_hbm.at[pl.ds(_N_PAD - _CLS_PER_TILE, ntail)],
                        prot_v.at[pl.ds(0, ntail)])

    def class_body(cl, win_lo):
        c = cbase + cl
        n_c = _sget(tot_v, c)
        start_c = _sget(excl_v, c)
        p = tuple(prot_v[cl, pl.ds(k * 16, 16)] for k in range(8))

        def samp_body(j, carry):
            wlo = carry[0]
            pk = carry[1:]
            row = start_c + j

            def reload(_):
                nl = (row // _WIN) * _WIN
                pltpu.sync_copy(sortedidx_sh.at[pl.ds(nl, _WIN)], idxwin_v)
                pltpu.async_copy(feat_hbm.at[idxwin_v], featwin_v, sem).wait()
                return nl

            wlo = lax.cond(row >= wlo + _WIN, reload, lambda _: wlo, 0)
            off = row - wlo
            f = tuple(featwin_v[off, pl.ds(k * 16, 16)] for k in range(8))
            new = tuple(pk[k] * _M + f[k] * (1.0 - _M) for k in range(8))
            acc = new[0] * new[0]
            for k in range(1, 8):
                acc = acc + new[k] * new[k]
            ss = jnp.maximum(_allsum16(acc, iota16)[0], jnp.float32(1e-24))
            yi = jnp.int32(0x5F3759DF) - lax.shift_right_logical(
                lax.bitcast_convert_type(ss, jnp.int32), 1)
            y = lax.bitcast_convert_type(yi, jnp.float32)
            for _ in range(3):
                y = y * (1.5 - 0.5 * ss * y * y)
            y = jnp.minimum(y, jnp.float32(1e12))
            yv = jnp.full((16,), y, jnp.float32)
            new = tuple(n * yv for n in new)
            return (wlo,) + new

        carry = lax.fori_loop(0, n_c, samp_body, (win_lo,) + p)
        for k in range(8):
            prot_v[cl, pl.ds(k * 16, 16)] = carry[1 + k]
        return carry[0]

    lax.fori_loop(0, _CLS_PER_TILE, class_body, jnp.int32(-(2 ** 30)))

    _stk.close()
    @pl.when(cbase != _N_PAD - _CLS_PER_TILE)
    def _():
        pltpu.sync_copy(prot_v, out_hbm.at[pl.ds(cbase, _CLS_PER_TILE)])

    @pl.when(cbase == _N_PAD - _CLS_PER_TILE)
    def _():
        pltpu.sync_copy(prot_v.at[pl.ds(0, ntail)],
                        out_hbm.at[pl.ds(_N_PAD - _CLS_PER_TILE, ntail)])


@functools.partial(
    pl.kernel,
    out_type=jax.ShapeDtypeStruct((_N_CLS, _FEAT), jnp.float32),
    mesh=plsc.VectorSubcoreMesh(core_axis_name="c", subcore_axis_name="s",
                                num_cores=_NSC, num_subcores=_NTILE),
    scratch_types=[
        pltpu.VMEM_SHARED((_NTILE, _N_PAD), jnp.int32),
        pltpu.VMEM_SHARED((_BATCH + _WIN,), jnp.int32),
        pltpu.VMEM((_CHUNK + 128,), jnp.int32),
        pltpu.VMEM((_N_PAD + 128,), jnp.int32),
        pltpu.VMEM((_CHUNK + 128,), jnp.int32),
        pltpu.VMEM((_NTILE, _N_PAD), jnp.int32),
        pltpu.VMEM((_N_PAD + 128,), jnp.int32),
        pltpu.VMEM((_N_PAD + 128,), jnp.int32),
        pltpu.VMEM((_N_PAD + 128,), jnp.int32),
        pltpu.VMEM((_CHUNK // 128, 128), jnp.int32),
        pltpu.VMEM((_CHUNK // 128, 128), jnp.int32),
        pltpu.VMEM((_WIN,), jnp.int32),
        pltpu.VMEM((_WIN, _FEAT), jnp.float32),
        pltpu.VMEM((_CLS_PER_TILE, _FEAT), jnp.float32),
        pltpu.VMEM((_WIN,), jnp.int32),
        pltpu.SemaphoreType.DMA,
    ],
)
def _sc_update(feat_hbm, lbl_hbm, prot_hbm, out_hbm, *scratch):
    _sc_body(feat_hbm, lbl_hbm, prot_hbm, out_hbm, *scratch)


def _loss_body(proto_ref, out_ref):
    p = proto_ref[...]
    logits = jax.lax.dot_general(
        p, p, (((1,), (1,)), ((), ())),
        preferred_element_type=jnp.float32) * (1.0 / _TEMP)
    row = jax.lax.broadcasted_iota(jnp.int32, (_N_CLS, _N_CLS), 0)
    col = jax.lax.broadcasted_iota(jnp.int32, (_N_CLS, _N_CLS), 1)
    offdiag = (row != col).astype(jnp.float32)
    num = jnp.sum(jnp.exp(logits) * offdiag, axis=1)
    mean_prob_neg = jnp.log(num * (1.0 / (_N_CLS - 1)))
    out_ref[0, 0] = jnp.sum(mean_prob_neg) * (1.0 / _N_CLS)


def kernel(features, labels, prototypes):
    updated = _sc_update(features, labels.astype(jnp.int32), prototypes)

    loss = pl.pallas_call(
        _loss_body,
        out_shape=jax.ShapeDtypeStruct((1, 1), jnp.float32),
        in_specs=[pl.BlockSpec(memory_space=pltpu.VMEM)],
        out_specs=pl.BlockSpec(memory_space=pltpu.SMEM),
    )(updated)
    return loss[0, 0]
